# baseline (speedup 1.0000x reference)
import jax
import jax.numpy as jnp
import numpy as np
from jax.experimental import pallas as pl
from jax.experimental.pallas import tpu as pltpu

_B = 128
_V = 100000
_BC = 2048
_NBLK = (_V + _BC - 1) // _BC

_KS0 = np.uint32(0)
_KS1 = np.uint32(42)
_KS2 = np.uint32(np.uint32(0) ^ np.uint32(42) ^ np.uint32(0x1BD11BDA))

_TINY = np.float32(np.finfo(np.float32).tiny)
_NEG_INF = np.float32(-np.inf)
_BIG_IDX = np.int32(0x7FFFFFFF)


def _rotl(x, d):
    return jax.lax.shift_left(x, np.uint32(d)) | jax.lax.shift_right_logical(
        x, np.uint32(32 - d)
    )


def _threefry_bits(lo):
    rot_a = (13, 15, 26, 6)
    rot_b = (17, 29, 16, 24)

    def rounds(x0, x1, rots):
        for r in rots:
            x0 = x0 + x1
            x1 = _rotl(x1, r)
            x1 = x1 ^ x0
        return x0, x1

    x1 = lo + _KS1
    x0 = x1
    x1 = _rotl(x1, 13) ^ x0
    x0, x1 = rounds(x0, x1, (15, 26, 6))
    x0 = x0 + _KS1
    x1 = x1 + np.uint32(_KS2 + np.uint32(1))
    x0, x1 = rounds(x0, x1, rot_b)
    x0 = x0 + _KS2
    x1 = x1 + np.uint32(_KS0 + np.uint32(2))
    x0, x1 = rounds(x0, x1, rot_a)
    x0 = x0 + _KS0
    x1 = x1 + np.uint32(_KS1 + np.uint32(3))
    x0, x1 = rounds(x0, x1, rot_b)
    x0 = x0 + _KS1
    x1 = x1 + np.uint32(_KS2 + np.uint32(4))
    x0, x1 = rounds(x0, x1, rot_a)
    x0 = x0 + _KS2
    x1 = x1 + np.uint32(_KS0 + np.uint32(5))
    return x0 ^ x1


def _body(logits_ref, out_ref, vmax_ref, vidx_ref):
    j = pl.program_id(0)

    x = logits_ref[...]
    col = jax.lax.broadcasted_iota(jnp.int32, (_B, _BC), 1) + j * _BC
    row = jax.lax.broadcasted_iota(jnp.int32, (_B, _BC), 0)
    flat = (row * _V + col).astype(jnp.uint32)

    bits = _threefry_bits(flat)
    fbits = jax.lax.shift_right_logical(bits, np.uint32(9)) | np.uint32(0x3F800000)
    flt = jax.lax.bitcast_convert_type(fbits, jnp.float32) - np.float32(1.0)
    u = jnp.maximum(flt, _TINY)
    g = -jnp.log(-jnp.log(u))
    val = jnp.where(col < _V, g + x, _NEG_INF)

    bm = jnp.max(val, axis=1, keepdims=True)
    bi = jnp.min(jnp.where(val == bm, col, _BIG_IDX), axis=1, keepdims=True)

    @pl.when(j == 0)
    def _():
        vmax_ref[...] = bm
        vidx_ref[...] = bi

    @pl.when(j > 0)
    def _():
        better = bm > vmax_ref[...]
        vmax_ref[...] = jnp.where(better, bm, vmax_ref[...])
        vidx_ref[...] = jnp.where(better, bi, vidx_ref[...])

    @pl.when(j == _NBLK - 1)
    def _():
        out_ref[...] = vidx_ref[...]


def kernel(logits):
    out = pl.pallas_call(
        _body,
        grid=(_NBLK,),
        in_specs=[pl.BlockSpec((_B, _BC), lambda j: (0, j))],
        out_specs=pl.BlockSpec((_B, 1), lambda j: (0, 0)),
        out_shape=jax.ShapeDtypeStruct((_B, 1), jnp.int32),
        scratch_shapes=[
            pltpu.VMEM((_B, 1), jnp.float32),
            pltpu.VMEM((_B, 1), jnp.int32),
        ],
        compiler_params=pltpu.CompilerParams(
            dimension_semantics=("arbitrary",),
        ),
    )(logits)
    return out.reshape(_B)
